# baseline (speedup 1.0000x reference)
import functools

import jax
import jax.numpy as jnp
from jax import lax
from jax.experimental import pallas as pl
from jax.experimental.pallas import tpu as pltpu
from jax.experimental.pallas import tpu_sc as plsc

_ENTRY = 8
_INDEX = 0
_R, _C = 16384, 128
_N = _R * _C
_SEED = 262144
_NC = 2
_PER_CORE = _N // _NC
_NDMA = _PER_CORE // _SEED


def _tc_seed_body(s_ref):
    ent = lax.rem(lax.broadcasted_iota(jnp.int32, s_ref.shape, 0), 2)
    lim = lax.broadcasted_iota(jnp.int32, s_ref.shape, 0) < _ENTRY
    val = jnp.sum(jnp.where(lim, ent, 0).astype(jnp.float32),
                  axis=0, keepdims=True)
    s_ref[...] = jnp.broadcast_to(val, s_ref.shape)


_tc_seed = pl.pallas_call(
    _tc_seed_body,
    out_shape=jax.ShapeDtypeStruct((_SEED // _C, _C), jnp.float32),
)


def _scs_cnt_body(seed_hbm, cnt_hbm, stage, sem):
    cid = lax.axis_index("c")
    base = cid * _PER_CORE
    pltpu.async_copy(seed_hbm, stage, sem).wait()

    def issue(j, carry):
        pltpu.async_copy(stage, cnt_hbm.at[pl.ds(base + j * _SEED, _SEED)], sem)
        return carry

    lax.fori_loop(0, _NDMA, issue, 0)

    def drain(j, carry):
        pltpu.make_async_copy(
            stage, cnt_hbm.at[pl.ds(base + j * _SEED, _SEED)], sem).wait()
        return carry

    lax.fori_loop(0, _NDMA, drain, 0)


_scs_cnt = functools.partial(
    pl.kernel,
    out_type=jax.ShapeDtypeStruct((_N,), jnp.float32),
    mesh=plsc.ScalarSubcoreMesh(axis_name="c"),
    scratch_types=[
        pltpu.MemorySpace.VMEM_SHARED((_SEED,), jnp.float32),
        pltpu.SemaphoreType.DMA,
    ],
)(_scs_cnt_body)


def _tc_out_body(o_ref):
    ent = lax.rem(lax.broadcasted_iota(jnp.int32, o_ref.shape, 0) + _INDEX, 2)
    sel = jnp.where(lax.broadcasted_iota(jnp.int32, o_ref.shape, 0) == 0,
                    ent, jnp.zeros_like(ent))
    col = jnp.max(sel, axis=0, keepdims=True)
    o_ref[...] = jnp.broadcast_to(col, o_ref.shape).astype(jnp.float32)


_TC_BLOCKS = 8
_tc_out = pl.pallas_call(
    _tc_out_body,
    out_shape=jax.ShapeDtypeStruct((_R, _C), jnp.float32),
    grid=(_TC_BLOCKS,),
    out_specs=pl.BlockSpec((_R // _TC_BLOCKS, _C), lambda i: (i, 0)),
)


def kernel(input):
    seed = _tc_seed().reshape(_SEED)
    cnt_flat = _scs_cnt(seed)
    out = _tc_out()
    return out, cnt_flat.reshape(_R, _C)


# speedup vs baseline: 1.1618x; 1.1618x over previous
import functools

import jax
import jax.numpy as jnp
from jax import lax
from jax.experimental import pallas as pl
from jax.experimental.pallas import tpu as pltpu
from jax.experimental.pallas import tpu_sc as plsc

_ENTRY = 8
_INDEX = 0
_R, _C = 16384, 128
_N = _R * _C
_NW = 32
_PER_W = _N // _NW
_BUF = 4096
_NDMA = _PER_W // _BUF


def _sc_cnt_body(cnt_hbm, cnt_buf, sem):
    cnt_val = lax.fori_loop(
        0, _ENTRY, lambda i, s: s + lax.rem(i, 2), jnp.int32(0))
    cnt_vec = jnp.broadcast_to(cnt_val.astype(jnp.float32), (16,))

    def store(i, carry):
        base = i * 128
        for u in range(8):
            cnt_buf[pl.ds(base + u * 16, 16)] = cnt_vec
        return carry

    lax.fori_loop(0, _BUF // 128, store, 0)

    wid = lax.axis_index("s") * 2 + lax.axis_index("c")
    base = wid * _PER_W
    copies = [
        pltpu.async_copy(cnt_buf, cnt_hbm.at[pl.ds(base + j * _BUF, _BUF)], sem)
        for j in range(_NDMA)
    ]
    for c in copies:
        c.wait()


_sc_cnt = functools.partial(
    pl.kernel,
    out_type=jax.ShapeDtypeStruct((_N,), jnp.float32),
    mesh=plsc.VectorSubcoreMesh(core_axis_name="c", subcore_axis_name="s"),
    scratch_types=[
        pltpu.VMEM((_BUF,), jnp.float32),
        pltpu.SemaphoreType.DMA,
    ],
)(_sc_cnt_body)


def _tc_out_body(o_ref):
    ent = lax.rem(lax.broadcasted_iota(jnp.int32, o_ref.shape, 0) + _INDEX, 2)
    sel = jnp.where(lax.broadcasted_iota(jnp.int32, o_ref.shape, 0) == 0,
                    ent, jnp.zeros_like(ent))
    col = jnp.max(sel, axis=0, keepdims=True)
    o_ref[...] = jnp.broadcast_to(col, o_ref.shape).astype(jnp.float32)


_TC_BLOCKS = 8
_tc_out = pl.pallas_call(
    _tc_out_body,
    out_shape=jax.ShapeDtypeStruct((_R, _C), jnp.float32),
    grid=(_TC_BLOCKS,),
    out_specs=pl.BlockSpec((_R // _TC_BLOCKS, _C), lambda i: (i, 0)),
)


def kernel(input):
    cnt_flat = _sc_cnt()
    out = _tc_out()
    return out, cnt_flat.reshape(_R, _C)
